# baseline (speedup 1.0000x reference)
import functools

import jax
import jax.numpy as jnp
from jax import lax
from jax.experimental import pallas as pl
from jax.experimental.pallas import tpu as pltpu
from jax.experimental.pallas import tpu_sc as plsc


def _sc_transpose(embT):
    D, V = embT.shape
    CW = 256
    NFULL = V // CW
    TAIL = V - NFULL * CW
    assert D % 16 == 0 and TAIL % 16 == 0 and TAIL < 128
    info = plsc.get_sparse_core_info()
    NW = info.num_cores * info.num_subcores
    CPT = (NFULL + NW - 1) // NW
    NC = info.num_cores

    mesh = plsc.VectorSubcoreMesh(core_axis_name="c", subcore_axis_name="s")

    @functools.partial(
        pl.kernel,
        out_type=jax.ShapeDtypeStruct((V * (D + 8),), jnp.float32),
        mesh=mesh,
        compiler_params=pltpu.CompilerParams(use_tc_tiling_on_sc=True,
                                             needs_layout_passes=False),
        scratch_types=[
            pltpu.VMEM((D // 8, 8, CW), jnp.float32),
            pltpu.VMEM((D // 8, 8, CW), jnp.float32),
            pltpu.VMEM((CW * (D + 8),), jnp.float32),
            pltpu.VMEM((CW * (D + 8),), jnp.float32),
            pltpu.SemaphoreType.DMA,
            pltpu.SemaphoreType.DMA,
            pltpu.SemaphoreType.DMA,
            pltpu.SemaphoreType.DMA,
        ],
    )
    def sc_tr(src_hbm, tail_hbm, out_hbm,
              in_v0, in_v1, out_v0, out_v1, isem0, isem1, osem0, osem1):
        wid = lax.axis_index("s") * NC + lax.axis_index("c")
        start = wid * CPT
        n = jnp.minimum(CPT, NFULL - start)
        iota = jnp.arange(16, dtype=jnp.int32)
        lane = iota * (D + 8)
        in_bufs = (in_v0, in_v1)
        out_bufs = (out_v0, out_v1)
        isems = (isem0, isem1)
        osems = (osem0, osem1)

        def in_src(k):
            v0 = pl.multiple_of((start + k) * CW, CW)
            return src_hbm.at[:, :, pl.ds(v0, CW)]

        def out_dst(k):
            o0 = pl.multiple_of((start + k) * CW * (D + 8), CW * (D + 8))
            return out_hbm.at[pl.ds(o0, CW * (D + 8))]

        pltpu.async_copy(in_src(0), in_v0, isem0)
        pltpu.async_copy(in_src(1), in_v1, isem1)

        def pair_body(p, carry):
            for b in range(2):
                k = p * 2 + b

                @pl.when(k < n)
                def _():
                    iv, ov = in_bufs[b], out_bufs[b]
                    pltpu.make_async_copy(in_src(k), iv, isems[b]).wait()

                    @pl.when(k >= 2)
                    def _():
                        pltpu.make_async_copy(ov, out_dst(k - 2),
                                              osems[b]).wait()

                    @plsc.parallel_loop(0, D, unroll=8)
                    def d_body(d):
                        tr = d // 8
                        s = d - tr * 8
                        for l16 in range(CW // 16):
                            x = iv[tr, s, pl.ds(l16 * 16, 16)]
                            idx = lane + (l16 * 16 * (D + 8) + d)
                            plsc.store_scatter(ov, [idx], x)
                    pltpu.async_copy(ov, out_dst(k), osems[b])

                    @pl.when(k + 2 < n)
                    def _():
                        pltpu.async_copy(in_src(k + 2), iv, isems[b])
            return carry

        lax.fori_loop(0, (CPT + 1) // 2, pair_body, 0)

        last_parity = (n - 1) % 2
        for b in range(2):
            m = jnp.where(last_parity == b, n - 1, n - 2)
            pltpu.make_async_copy(out_bufs[b], out_dst(m), osems[b]).wait()

        if TAIL:
            @pl.when(wid == NW - 1)
            def _():
                pltpu.sync_copy(tail_hbm, out_v0.at[pl.ds(0, TAIL * (D + 8))])
                pltpu.sync_copy(out_v0.at[pl.ds(0, TAIL * (D + 8))],
                                out_hbm.at[pl.ds(NFULL * CW * (D + 8),
                                                 TAIL * (D + 8))])

    tail_rows = embT[:, NFULL * CW:].T
    tail_flat = jnp.pad(tail_rows, ((0, 0), (0, 8))).reshape(-1)
    return sc_tr(embT.reshape(D // 8, 8, V), tail_flat)


def _sc_gather_sum(input_text, emb, D):
    B, L = input_text.shape
    V, DP = emb.shape
    info = plsc.get_sparse_core_info()
    NC, NS = info.num_cores, info.num_subcores
    NW = NC * NS
    assert B % NW == 0
    TPB = B // NW
    CH0 = min(128, L)
    CH1 = L - CH0
    assert CH0 % 8 == 0 and (CH1 == 0 or CH1 % 8 == 0)
    NV = D // 16

    mesh = plsc.VectorSubcoreMesh(core_axis_name="c", subcore_axis_name="s")

    assert TPB % 2 == 0

    @functools.partial(
        pl.kernel,
        out_type=jax.ShapeDtypeStruct((B, D), jnp.float32),
        mesh=mesh,
        compiler_params=pltpu.CompilerParams(use_tc_tiling_on_sc=False),
        scratch_types=[
            pltpu.VMEM((TPB, L), jnp.int32),
            pltpu.VMEM((L, DP), jnp.float32),
            pltpu.VMEM((L, DP), jnp.float32),
            pltpu.VMEM((TPB, D), jnp.float32),
            pltpu.SemaphoreType.DMA,
            pltpu.SemaphoreType.DMA,
        ],
    )
    def sc_sum(text_hbm, emb_hbm, out_hbm, idx_v, rows0, rows1, out_v,
               sem0, sem1):
        wid = lax.axis_index("s") * NC + lax.axis_index("c")
        base = pl.multiple_of(wid * TPB, 8)
        pltpu.sync_copy(text_hbm.at[pl.ds(base, TPB)], idx_v)
        bufs = (rows0, rows1)
        sems = (sem0, sem1)

        def copies(t, buf, sem):
            yield pltpu.make_async_copy(
                emb_hbm.at[idx_v.at[t, pl.ds(0, CH0)]],
                buf.at[pl.ds(0, CH0)], sem)
            if CH1:
                yield pltpu.make_async_copy(
                    emb_hbm.at[idx_v.at[t, pl.ds(CH0, CH1)]],
                    buf.at[pl.ds(CH0, CH1)], sem)

        for c in copies(0, rows0, sem0):
            c.start()
        for c in copies(1, rows1, sem1):
            c.start()

        def pair_body(p, carry):
            for b in range(2):
                t = p * 2 + b
                buf, sem = bufs[b], sems[b]
                for c in copies(t, buf, sem):
                    c.wait()

                zeros = tuple(jnp.zeros((16,), jnp.float32)
                              for _ in range(NV))

                @plsc.parallel_loop(0, L, unroll=8, carry=zeros)
                def acc(j, c):
                    return tuple(c[q] + buf[j, pl.ds(q * 16, 16)]
                                 for q in range(NV))
                for q in range(NV):
                    out_v[t, pl.ds(q * 16, 16)] = acc[q]

                @pl.when(t + 2 < TPB)
                def _():
                    for c in copies(t + 2, buf, sem):
                        c.start()
            return carry

        lax.fori_loop(0, TPB // 2, pair_body, 0)
        pltpu.sync_copy(out_v, out_hbm.at[pl.ds(base, TPB)])

    return sc_sum(input_text, emb)


def _mlp_body(sum_ref, len_ref, w1_ref, b1_ref, w2_ref, b2_ref, out_ref):
    avg = sum_ref[...] / len_ref[...]
    h = lax.dot_general(avg, w1_ref[...], (((1,), (1,)), ((), ())),
                        preferred_element_type=jnp.float32) + b1_ref[...]
    h = jnp.maximum(h, 0.0)
    out_ref[...] = lax.dot_general(h, w2_ref[...], (((1,), (1,)), ((), ())),
                                   preferred_element_type=jnp.float32) + b2_ref[...]


def _mlp(summed, lenf, W1, b1, W2, b2):
    B, D = summed.shape
    H = W1.shape[0]
    C = W2.shape[0]
    BT = 512
    grid = (B // BT,)
    return pl.pallas_call(
        _mlp_body,
        grid=grid,
        in_specs=[
            pl.BlockSpec((BT, D), lambda i: (i, 0)),
            pl.BlockSpec((BT, 1), lambda i: (i, 0)),
            pl.BlockSpec((H, D), lambda i: (0, 0)),
            pl.BlockSpec((1, H), lambda i: (0, 0)),
            pl.BlockSpec((C, H), lambda i: (0, 0)),
            pl.BlockSpec((1, C), lambda i: (0, 0)),
        ],
        out_specs=pl.BlockSpec((BT, C), lambda i: (i, 0)),
        out_shape=jax.ShapeDtypeStruct((B, C), jnp.float32),
    )(summed, lenf, W1, b1.reshape(1, H), W2, b2.reshape(1, C))


def kernel(input_text, text_len, emb, W1, b1, W2, b2):
    B = input_text.shape[0]
    V, D = emb.shape
    table = _sc_transpose(emb.T).reshape(V, D + 8)
    summed = _sc_gather_sum(input_text, table, D)
    lenf = text_len.astype(jnp.float32).reshape(B, 1)
    return _mlp(summed, lenf, W1, b1, W2, b2)


# speedup vs baseline: 1.2457x; 1.2457x over previous
import functools

import jax
import jax.numpy as jnp
from jax import lax
from jax.experimental import pallas as pl
from jax.experimental.pallas import tpu as pltpu
from jax.experimental.pallas import tpu_sc as plsc


def _sc_transpose(embT):
    D, V = embT.shape
    PW = D // 2 + 8
    CW = 256
    NFULL = V // CW
    TAIL = V - NFULL * CW
    assert D % 16 == 0 and TAIL % 16 == 0 and TAIL < 128
    info = plsc.get_sparse_core_info()
    NW = info.num_cores * info.num_subcores
    CPT = (NFULL + NW - 1) // NW
    NC = info.num_cores

    mesh = plsc.VectorSubcoreMesh(core_axis_name="c", subcore_axis_name="s")

    @functools.partial(
        pl.kernel,
        out_type=jax.ShapeDtypeStruct((V * PW,), jnp.int32),
        mesh=mesh,
        compiler_params=pltpu.CompilerParams(use_tc_tiling_on_sc=True,
                                             needs_layout_passes=False),
        scratch_types=[
            pltpu.VMEM((D // 8, 8, CW), jnp.float32),
            pltpu.VMEM((D // 8, 8, CW), jnp.float32),
            pltpu.VMEM((CW * PW,), jnp.int32),
            pltpu.VMEM((CW * PW,), jnp.int32),
            pltpu.SemaphoreType.DMA,
            pltpu.SemaphoreType.DMA,
            pltpu.SemaphoreType.DMA,
            pltpu.SemaphoreType.DMA,
        ],
    )
    def sc_tr(src_hbm, tail_hbm, out_hbm,
              in_v0, in_v1, out_v0, out_v1, isem0, isem1, osem0, osem1):
        wid = lax.axis_index("s") * NC + lax.axis_index("c")
        start = wid * CPT
        n = jnp.minimum(CPT, NFULL - start)
        iota = jnp.arange(16, dtype=jnp.int32)
        lane = iota * PW
        in_bufs = (in_v0, in_v1)
        out_bufs = (out_v0, out_v1)
        isems = (isem0, isem1)
        osems = (osem0, osem1)

        def in_src(k):
            v0 = pl.multiple_of((start + k) * CW, CW)
            return src_hbm.at[:, :, pl.ds(v0, CW)]

        def out_dst(k):
            o0 = pl.multiple_of((start + k) * CW * PW, CW * PW)
            return out_hbm.at[pl.ds(o0, CW * PW)]

        pltpu.async_copy(in_src(0), in_v0, isem0)
        pltpu.async_copy(in_src(1), in_v1, isem1)

        def pair_body(p, carry):
            for b in range(2):
                k = p * 2 + b

                @pl.when(k < n)
                def _():
                    iv, ov = in_bufs[b], out_bufs[b]
                    pltpu.make_async_copy(in_src(k), iv, isems[b]).wait()

                    @pl.when(k >= 2)
                    def _():
                        pltpu.make_async_copy(ov, out_dst(k - 2),
                                              osems[b]).wait()

                    @plsc.parallel_loop(0, D // 2, unroll=8)
                    def m_body(m):
                        tr = m // 4
                        s = 2 * (m - tr * 4)
                        for l16 in range(CW // 16):
                            a = iv[tr, s, pl.ds(l16 * 16, 16)]
                            b = iv[tr, s + 1, pl.ds(l16 * 16, 16)]
                            y = plsc.pack(a, b,
                                          format=plsc.PackFormat.INTERLEAVED)
                            w = plsc.bitcast(y, jnp.int32)
                            idx = lane + (l16 * 16 * PW + m)
                            plsc.store_scatter(ov, [idx], w)
                    pltpu.async_copy(ov, out_dst(k), osems[b])

                    @pl.when(k + 2 < n)
                    def _():
                        pltpu.async_copy(in_src(k + 2), iv, isems[b])
            return carry

        lax.fori_loop(0, (CPT + 1) // 2, pair_body, 0)

        last_parity = (n - 1) % 2
        for b in range(2):
            m = jnp.where(last_parity == b, n - 1, n - 2)
            pltpu.make_async_copy(out_bufs[b], out_dst(m), osems[b]).wait()

        if TAIL:
            @pl.when(wid == NW - 1)
            def _():
                pltpu.sync_copy(tail_hbm, out_v0.at[pl.ds(0, TAIL * PW)])
                pltpu.sync_copy(out_v0.at[pl.ds(0, TAIL * PW)],
                                out_hbm.at[pl.ds(NFULL * CW * PW,
                                                 TAIL * PW)])

    tail_bf = embT[:, NFULL * CW:].T.astype(jnp.bfloat16)
    tail_bf = jnp.pad(tail_bf, ((0, 0), (0, 2 * PW - D))).reshape(TAIL, PW, 2)
    tail_flat = jax.lax.bitcast_convert_type(tail_bf, jnp.int32).reshape(-1)
    return sc_tr(embT.reshape(D // 8, 8, V), tail_flat)


def _sc_gather_sum(input_text, emb, D):
    B, L = input_text.shape
    V, PW = emb.shape
    info = plsc.get_sparse_core_info()
    NC, NS = info.num_cores, info.num_subcores
    NW = NC * NS
    assert B % NW == 0
    TPB = B // NW
    CH0 = min(128, L)
    CH1 = L - CH0
    assert CH0 % 8 == 0 and (CH1 == 0 or CH1 % 8 == 0)
    NG = D // 32

    mesh = plsc.VectorSubcoreMesh(core_axis_name="c", subcore_axis_name="s")

    assert TPB % 2 == 0

    @functools.partial(
        pl.kernel,
        out_type=jax.ShapeDtypeStruct((B, D), jnp.float32),
        mesh=mesh,
        compiler_params=pltpu.CompilerParams(use_tc_tiling_on_sc=False,
                                             needs_layout_passes=False),
        scratch_types=[
            pltpu.VMEM((TPB, L), jnp.int32),
            pltpu.VMEM((L, PW), jnp.int32),
            pltpu.VMEM((L, PW), jnp.int32),
            pltpu.VMEM((TPB, D), jnp.float32),
            pltpu.SemaphoreType.DMA,
            pltpu.SemaphoreType.DMA,
        ],
    )
    def sc_sum(text_hbm, emb_hbm, out_hbm, idx_v, rows0, rows1, out_v,
               sem0, sem1):
        wid = lax.axis_index("s") * NC + lax.axis_index("c")
        base = pl.multiple_of(wid * TPB, 8)
        pltpu.sync_copy(text_hbm.at[pl.ds(base, TPB)], idx_v)
        bufs = (rows0, rows1)
        sems = (sem0, sem1)

        def copies(t, buf, sem):
            yield pltpu.make_async_copy(
                emb_hbm.at[idx_v.at[t, pl.ds(0, CH0)]],
                buf.at[pl.ds(0, CH0)], sem)
            if CH1:
                yield pltpu.make_async_copy(
                    emb_hbm.at[idx_v.at[t, pl.ds(CH0, CH1)]],
                    buf.at[pl.ds(CH0, CH1)], sem)

        for c in copies(0, rows0, sem0):
            c.start()
        for c in copies(1, rows1, sem1):
            c.start()

        def pair_body(p, carry):
            for b in range(2):
                t = p * 2 + b
                buf, sem = bufs[b], sems[b]
                for c in copies(t, buf, sem):
                    c.wait()

                def row_body(j, acc):
                    new = []
                    for g in range(NG):
                        w = buf[j, pl.ds(g * 16, 16)]
                        y = plsc.bitcast(w, jnp.bfloat16)
                        ev, od = plsc.unpack(
                            y, format=plsc.PackFormat.INTERLEAVED,
                            preferred_element_type=jnp.float32)
                        new.append(acc[2 * g] + ev)
                        new.append(acc[2 * g + 1] + od)
                    return tuple(new)

                acc = lax.fori_loop(
                    0, L, row_body,
                    tuple(jnp.zeros((16,), jnp.float32)
                          for _ in range(2 * NG)),
                    unroll=8)
                iota = jnp.arange(16, dtype=jnp.int32)
                trow = t + jnp.zeros((16,), jnp.int32)
                for g in range(NG):
                    for r in range(2):
                        cidx = 2 * iota + (32 * g + r)
                        plsc.store_scatter(out_v, [trow, cidx],
                                           acc[2 * g + r])

                @pl.when(t + 2 < TPB)
                def _():
                    for c in copies(t + 2, buf, sem):
                        c.start()
            return carry

        lax.fori_loop(0, TPB // 2, pair_body, 0)
        pltpu.sync_copy(out_v, out_hbm.at[pl.ds(base, TPB)])

    return sc_sum(input_text, emb)


def _mlp_body(sum_ref, len_ref, w1_ref, b1_ref, w2_ref, b2_ref, out_ref):
    avg = sum_ref[...] / len_ref[...]
    h = lax.dot_general(avg, w1_ref[...], (((1,), (1,)), ((), ())),
                        preferred_element_type=jnp.float32) + b1_ref[...]
    h = jnp.maximum(h, 0.0)
    out_ref[...] = lax.dot_general(h, w2_ref[...], (((1,), (1,)), ((), ())),
                                   preferred_element_type=jnp.float32) + b2_ref[...]


def _mlp(summed, lenf, W1, b1, W2, b2):
    B, D = summed.shape
    H = W1.shape[0]
    C = W2.shape[0]
    BT = 512
    grid = (B // BT,)
    return pl.pallas_call(
        _mlp_body,
        grid=grid,
        in_specs=[
            pl.BlockSpec((BT, D), lambda i: (i, 0)),
            pl.BlockSpec((BT, 1), lambda i: (i, 0)),
            pl.BlockSpec((H, D), lambda i: (0, 0)),
            pl.BlockSpec((1, H), lambda i: (0, 0)),
            pl.BlockSpec((C, H), lambda i: (0, 0)),
            pl.BlockSpec((1, C), lambda i: (0, 0)),
        ],
        out_specs=pl.BlockSpec((BT, C), lambda i: (i, 0)),
        out_shape=jax.ShapeDtypeStruct((B, C), jnp.float32),
    )(summed, lenf, W1, b1.reshape(1, H), W2, b2.reshape(1, C))


def kernel(input_text, text_len, emb, W1, b1, W2, b2):
    B = input_text.shape[0]
    V, D = emb.shape
    table = _sc_transpose(emb.T).reshape(V, D // 2 + 8)
    summed = _sc_gather_sum(input_text, table, D)
    lenf = text_len.astype(jnp.float32).reshape(B, 1)
    return _mlp(summed, lenf, W1, b1, W2, b2)


# speedup vs baseline: 1.4444x; 1.1595x over previous
import functools

import jax
import jax.numpy as jnp
from jax import lax
from jax.experimental import pallas as pl
from jax.experimental.pallas import tpu as pltpu
from jax.experimental.pallas import tpu_sc as plsc


def _sc_transpose(embT):
    D, V = embT.shape
    PW = D // 2 + 8
    CW = 512
    NFULL = V // CW
    TAIL = V - NFULL * CW
    assert D % 16 == 0 and TAIL % 16 == 0 and TAIL < 128
    info = plsc.get_sparse_core_info()
    NW = info.num_cores * info.num_subcores
    CPT = (NFULL + NW - 1) // NW
    NC = info.num_cores

    mesh = plsc.VectorSubcoreMesh(core_axis_name="c", subcore_axis_name="s")

    @functools.partial(
        pl.kernel,
        out_type=jax.ShapeDtypeStruct((V * PW,), jnp.int32),
        mesh=mesh,
        compiler_params=pltpu.CompilerParams(use_tc_tiling_on_sc=True,
                                             needs_layout_passes=False),
        scratch_types=[
            pltpu.VMEM((D // 8, 8, CW), jnp.float32),
            pltpu.VMEM((D // 8, 8, CW), jnp.float32),
            pltpu.VMEM((CW * PW,), jnp.int32),
            pltpu.VMEM((CW * PW,), jnp.int32),
            pltpu.SemaphoreType.DMA,
            pltpu.SemaphoreType.DMA,
            pltpu.SemaphoreType.DMA,
            pltpu.SemaphoreType.DMA,
        ],
    )
    def sc_tr(src_hbm, tail_hbm, out_hbm,
              in_v0, in_v1, out_v0, out_v1, isem0, isem1, osem0, osem1):
        wid = lax.axis_index("s") * NC + lax.axis_index("c")
        start = wid * CPT
        n = jnp.minimum(CPT, NFULL - start)
        iota = jnp.arange(16, dtype=jnp.int32)
        lane = iota * PW
        in_bufs = (in_v0, in_v1)
        out_bufs = (out_v0, out_v1)
        isems = (isem0, isem1)
        osems = (osem0, osem1)

        def in_src(k):
            v0 = pl.multiple_of((start + k) * CW, CW)
            return src_hbm.at[:, :, pl.ds(v0, CW)]

        def out_dst(k):
            o0 = pl.multiple_of((start + k) * CW * PW, CW * PW)
            return out_hbm.at[pl.ds(o0, CW * PW)]

        pltpu.async_copy(in_src(0), in_v0, isem0)
        pltpu.async_copy(in_src(1), in_v1, isem1)

        def pair_body(p, carry):
            for b in range(2):
                k = p * 2 + b

                @pl.when(k < n)
                def _():
                    iv, ov = in_bufs[b], out_bufs[b]
                    pltpu.make_async_copy(in_src(k), iv, isems[b]).wait()

                    @pl.when(k >= 2)
                    def _():
                        pltpu.make_async_copy(ov, out_dst(k - 2),
                                              osems[b]).wait()

                    @plsc.parallel_loop(0, D // 2, unroll=8)
                    def m_body(m):
                        tr = m // 4
                        s = 2 * (m - tr * 4)
                        for l16 in range(CW // 16):
                            a = iv[tr, s, pl.ds(l16 * 16, 16)]
                            b = iv[tr, s + 1, pl.ds(l16 * 16, 16)]
                            y = plsc.pack(a, b,
                                          format=plsc.PackFormat.INTERLEAVED)
                            w = plsc.bitcast(y, jnp.int32)
                            idx = lane + (l16 * 16 * PW + m)
                            plsc.store_scatter(ov, [idx], w)
                    pltpu.async_copy(ov, out_dst(k), osems[b])

                    @pl.when(k + 2 < n)
                    def _():
                        pltpu.async_copy(in_src(k + 2), iv, isems[b])
            return carry

        lax.fori_loop(0, (CPT + 1) // 2, pair_body, 0)

        last_parity = (n - 1) % 2
        for b in range(2):
            m = jnp.where(last_parity == b, n - 1, n - 2)
            pltpu.make_async_copy(out_bufs[b], out_dst(m), osems[b]).wait()

        if TAIL:
            @pl.when(wid == NW - 1)
            def _():
                pltpu.sync_copy(tail_hbm, out_v0.at[pl.ds(0, TAIL * PW)])
                pltpu.sync_copy(out_v0.at[pl.ds(0, TAIL * PW)],
                                out_hbm.at[pl.ds(NFULL * CW * PW,
                                                 TAIL * PW)])

    tail_bf = embT[:, NFULL * CW:].T.astype(jnp.bfloat16)
    tail_bf = jnp.pad(tail_bf, ((0, 0), (0, 2 * PW - D))).reshape(TAIL, PW, 2)
    tail_flat = jax.lax.bitcast_convert_type(tail_bf, jnp.int32).reshape(-1)
    return sc_tr(embT.reshape(D // 8, 8, V), tail_flat)


def _sc_gather_sum(input_text, emb, D):
    B, L = input_text.shape
    V, PW = emb.shape
    info = plsc.get_sparse_core_info()
    NC, NS = info.num_cores, info.num_subcores
    NW = NC * NS
    assert B % NW == 0
    TPB = B // NW
    CH0 = min(128, L)
    CH1 = L - CH0
    assert CH0 % 8 == 0 and (CH1 == 0 or CH1 % 8 == 0)
    NG = D // 32

    mesh = plsc.VectorSubcoreMesh(core_axis_name="c", subcore_axis_name="s")

    assert TPB % 4 == 0

    @functools.partial(
        pl.kernel,
        out_type=jax.ShapeDtypeStruct((B, D), jnp.float32),
        mesh=mesh,
        compiler_params=pltpu.CompilerParams(use_tc_tiling_on_sc=False,
                                             needs_layout_passes=False),
        scratch_types=[
            pltpu.VMEM((TPB, L), jnp.int32),
            pltpu.VMEM((L, PW), jnp.int32),
            pltpu.VMEM((L, PW), jnp.int32),
            pltpu.VMEM((L, PW), jnp.int32),
            pltpu.VMEM((L, PW), jnp.int32),
            pltpu.VMEM((TPB, D), jnp.float32),
            pltpu.SemaphoreType.DMA,
            pltpu.SemaphoreType.DMA,
            pltpu.SemaphoreType.DMA,
            pltpu.SemaphoreType.DMA,
        ],
    )
    def sc_sum(text_hbm, emb_hbm, out_hbm, idx_v, rows0, rows1, rows2, rows3,
               out_v, sem0, sem1, sem2, sem3):
        wid = lax.axis_index("s") * NC + lax.axis_index("c")
        base = pl.multiple_of(wid * TPB, 8)
        pltpu.sync_copy(text_hbm.at[pl.ds(base, TPB)], idx_v)
        bufs = (rows0, rows1, rows2, rows3)
        sems = (sem0, sem1, sem2, sem3)

        def copies(t, buf, sem):
            yield pltpu.make_async_copy(
                emb_hbm.at[idx_v.at[t, pl.ds(0, CH0)]],
                buf.at[pl.ds(0, CH0)], sem)
            if CH1:
                yield pltpu.make_async_copy(
                    emb_hbm.at[idx_v.at[t, pl.ds(CH0, CH1)]],
                    buf.at[pl.ds(CH0, CH1)], sem)

        for tp in range(4):
            for c in copies(tp, bufs[tp], sems[tp]):
                c.start()

        def pair_body(p, carry):
            for b in range(4):
                t = p * 4 + b
                buf, sem = bufs[b], sems[b]
                for c in copies(t, buf, sem):
                    c.wait()

                def row_body(j, acc):
                    new = []
                    for g in range(NG):
                        w = buf[j, pl.ds(g * 16, 16)]
                        y = plsc.bitcast(w, jnp.bfloat16)
                        ev, od = plsc.unpack(
                            y, format=plsc.PackFormat.INTERLEAVED,
                            preferred_element_type=jnp.float32)
                        new.append(acc[2 * g] + ev)
                        new.append(acc[2 * g + 1] + od)
                    return tuple(new)

                acc = lax.fori_loop(
                    0, L, row_body,
                    tuple(jnp.zeros((16,), jnp.float32)
                          for _ in range(2 * NG)),
                    unroll=8)
                iota = jnp.arange(16, dtype=jnp.int32)
                trow = t + jnp.zeros((16,), jnp.int32)
                for g in range(NG):
                    for r in range(2):
                        cidx = 2 * iota + (32 * g + r)
                        plsc.store_scatter(out_v, [trow, cidx],
                                           acc[2 * g + r])

                @pl.when(t + 4 < TPB)
                def _():
                    for c in copies(t + 4, buf, sem):
                        c.start()
            return carry

        lax.fori_loop(0, TPB // 4, pair_body, 0)
        pltpu.sync_copy(out_v, out_hbm.at[pl.ds(base, TPB)])

    return sc_sum(input_text, emb)


def _mlp_body(sum_ref, len_ref, w1_ref, b1_ref, w2_ref, b2_ref, out_ref):
    avg = sum_ref[...] / len_ref[...]
    h = lax.dot_general(avg, w1_ref[...], (((1,), (1,)), ((), ())),
                        preferred_element_type=jnp.float32) + b1_ref[...]
    h = jnp.maximum(h, 0.0)
    out_ref[...] = lax.dot_general(h, w2_ref[...], (((1,), (1,)), ((), ())),
                                   preferred_element_type=jnp.float32) + b2_ref[...]


def _mlp(summed, lenf, W1, b1, W2, b2):
    B, D = summed.shape
    H = W1.shape[0]
    C = W2.shape[0]
    BT = 512
    grid = (B // BT,)
    return pl.pallas_call(
        _mlp_body,
        grid=grid,
        in_specs=[
            pl.BlockSpec((BT, D), lambda i: (i, 0)),
            pl.BlockSpec((BT, 1), lambda i: (i, 0)),
            pl.BlockSpec((H, D), lambda i: (0, 0)),
            pl.BlockSpec((1, H), lambda i: (0, 0)),
            pl.BlockSpec((C, H), lambda i: (0, 0)),
            pl.BlockSpec((1, C), lambda i: (0, 0)),
        ],
        out_specs=pl.BlockSpec((BT, C), lambda i: (i, 0)),
        out_shape=jax.ShapeDtypeStruct((B, C), jnp.float32),
    )(summed, lenf, W1, b1.reshape(1, H), W2, b2.reshape(1, C))


def kernel(input_text, text_len, emb, W1, b1, W2, b2):
    B = input_text.shape[0]
    V, D = emb.shape
    table = _sc_transpose(emb.T).reshape(V, D // 2 + 8)
    summed = _sc_gather_sum(input_text, table, D)
    lenf = text_len.astype(jnp.float32).reshape(B, 1)
    return _mlp(summed, lenf, W1, b1, W2, b2)


# speedup vs baseline: 1.4476x; 1.0022x over previous
import functools

import jax
import jax.numpy as jnp
from jax import lax
from jax.experimental import pallas as pl
from jax.experimental.pallas import tpu as pltpu
from jax.experimental.pallas import tpu_sc as plsc


def _sc_transpose(embT):
    D, V = embT.shape
    PW = D // 2 + 8
    CW = 512
    NFULL = V // CW
    TAIL = V - NFULL * CW
    assert D % 16 == 0 and TAIL % 16 == 0 and TAIL < 128
    info = plsc.get_sparse_core_info()
    NW = info.num_cores * info.num_subcores
    CPT = (NFULL + NW - 1) // NW
    NC = info.num_cores

    mesh = plsc.VectorSubcoreMesh(core_axis_name="c", subcore_axis_name="s")

    @functools.partial(
        pl.kernel,
        out_type=jax.ShapeDtypeStruct((V * PW,), jnp.int32),
        mesh=mesh,
        compiler_params=pltpu.CompilerParams(use_tc_tiling_on_sc=True,
                                             needs_layout_passes=False),
        scratch_types=[
            pltpu.VMEM((D // 8, 8, CW), jnp.float32),
            pltpu.VMEM((D // 8, 8, CW), jnp.float32),
            pltpu.VMEM((CW * PW,), jnp.int32),
            pltpu.VMEM((CW * PW,), jnp.int32),
            pltpu.SemaphoreType.DMA,
            pltpu.SemaphoreType.DMA,
            pltpu.SemaphoreType.DMA,
            pltpu.SemaphoreType.DMA,
        ],
    )
    def sc_tr(src_hbm, tail_hbm, out_hbm,
              in_v0, in_v1, out_v0, out_v1, isem0, isem1, osem0, osem1):
        wid = lax.axis_index("s") * NC + lax.axis_index("c")
        start = wid * CPT
        n = jnp.minimum(CPT, NFULL - start)
        iota = jnp.arange(16, dtype=jnp.int32)
        lane = iota * PW
        in_bufs = (in_v0, in_v1)
        out_bufs = (out_v0, out_v1)
        isems = (isem0, isem1)
        osems = (osem0, osem1)

        def in_src(k):
            v0 = pl.multiple_of((start + k) * CW, CW)
            return src_hbm.at[:, :, pl.ds(v0, CW)]

        def out_dst(k):
            o0 = pl.multiple_of((start + k) * CW * PW, CW * PW)
            return out_hbm.at[pl.ds(o0, CW * PW)]

        pltpu.async_copy(in_src(0), in_v0, isem0)
        pltpu.async_copy(in_src(1), in_v1, isem1)

        def pair_body(p, carry):
            for b in range(2):
                k = p * 2 + b

                @pl.when(k < n)
                def _():
                    iv, ov = in_bufs[b], out_bufs[b]
                    pltpu.make_async_copy(in_src(k), iv, isems[b]).wait()

                    @pl.when(k >= 2)
                    def _():
                        pltpu.make_async_copy(ov, out_dst(k - 2),
                                              osems[b]).wait()

                    @plsc.parallel_loop(0, D // 2, unroll=8)
                    def m_body(m):
                        tr = m // 4
                        s = 2 * (m - tr * 4)
                        for l16 in range(CW // 16):
                            a = iv[tr, s, pl.ds(l16 * 16, 16)]
                            b = iv[tr, s + 1, pl.ds(l16 * 16, 16)]
                            y = plsc.pack(a, b,
                                          format=plsc.PackFormat.INTERLEAVED)
                            w = plsc.bitcast(y, jnp.int32)
                            idx = lane + (l16 * 16 * PW + m)
                            plsc.store_scatter(ov, [idx], w)
                    pltpu.async_copy(ov, out_dst(k), osems[b])

                    @pl.when(k + 2 < n)
                    def _():
                        pltpu.async_copy(in_src(k + 2), iv, isems[b])
            return carry

        lax.fori_loop(0, (CPT + 1) // 2, pair_body, 0)

        last_parity = (n - 1) % 2
        for b in range(2):
            m = jnp.where(last_parity == b, n - 1, n - 2)
            pltpu.make_async_copy(out_bufs[b], out_dst(m), osems[b]).wait()

        if TAIL:
            @pl.when(wid == NW - 1)
            def _():
                pltpu.sync_copy(tail_hbm, out_v0.at[pl.ds(0, TAIL * PW)])
                pltpu.sync_copy(out_v0.at[pl.ds(0, TAIL * PW)],
                                out_hbm.at[pl.ds(NFULL * CW * PW,
                                                 TAIL * PW)])

    tail_bf = embT[:, NFULL * CW:].T.astype(jnp.bfloat16)
    tail_bf = jnp.pad(tail_bf, ((0, 0), (0, 2 * PW - D))).reshape(TAIL, PW, 2)
    tail_flat = jax.lax.bitcast_convert_type(tail_bf, jnp.int32).reshape(-1)
    return sc_tr(embT.reshape(D // 8, 8, V), tail_flat)


def _sc_gather_sum(text_flat, text_len, emb, B, L, D):
    V, PW = emb.shape
    info = plsc.get_sparse_core_info()
    NC, NS = info.num_cores, info.num_subcores
    NW = NC * NS
    assert B % NW == 0
    TPB = B // NW
    CH0 = min(128, L)
    CH1 = L - CH0
    assert CH0 % 8 == 0 and (CH1 == 0 or CH1 % 8 == 0)
    NG = D // 32

    mesh = plsc.VectorSubcoreMesh(core_axis_name="c", subcore_axis_name="s")

    assert TPB % 4 == 0

    @functools.partial(
        pl.kernel,
        out_type=jax.ShapeDtypeStruct((B, D), jnp.float32),
        mesh=mesh,
        compiler_params=pltpu.CompilerParams(use_tc_tiling_on_sc=False,
                                             needs_layout_passes=False),
        scratch_types=[
            pltpu.VMEM((TPB * L,), jnp.int32),
            pltpu.VMEM((TPB,), jnp.float32),
            pltpu.VMEM((L, PW), jnp.int32),
            pltpu.VMEM((L, PW), jnp.int32),
            pltpu.VMEM((L, PW), jnp.int32),
            pltpu.VMEM((L, PW), jnp.int32),
            pltpu.VMEM((TPB, D), jnp.float32),
            pltpu.SemaphoreType.DMA,
            pltpu.SemaphoreType.DMA,
            pltpu.SemaphoreType.DMA,
            pltpu.SemaphoreType.DMA,
        ],
    )
    def sc_sum(text_hbm, len_hbm, emb_hbm, out_hbm, idx_v, rlen_v,
               rows0, rows1, rows2, rows3, out_v, sem0, sem1, sem2, sem3):
        wid = lax.axis_index("s") * NC + lax.axis_index("c")
        base = pl.multiple_of(wid * TPB, 8)
        pltpu.sync_copy(text_hbm.at[pl.ds(pl.multiple_of(base * L, 8),
                                          TPB * L)], idx_v)
        pltpu.sync_copy(len_hbm.at[pl.ds(base, TPB)], rlen_v)
        bufs = (rows0, rows1, rows2, rows3)
        sems = (sem0, sem1, sem2, sem3)

        def copies(t, buf, sem):
            o0 = pl.multiple_of(t * L, 8)
            yield pltpu.make_async_copy(
                emb_hbm.at[idx_v.at[pl.ds(o0, CH0)]],
                buf.at[pl.ds(0, CH0)], sem)
            if CH1:
                yield pltpu.make_async_copy(
                    emb_hbm.at[idx_v.at[pl.ds(o0 + CH0, CH1)]],
                    buf.at[pl.ds(CH0, CH1)], sem)

        for tp in range(4):
            for c in copies(tp, bufs[tp], sems[tp]):
                c.start()

        def pair_body(p, carry):
            for b in range(4):
                t = p * 4 + b
                buf, sem = bufs[b], sems[b]
                for c in copies(t, buf, sem):
                    c.wait()

                def row_body(j, acc):
                    new = []
                    for g in range(NG):
                        w = buf[j, pl.ds(g * 16, 16)]
                        y = plsc.bitcast(w, jnp.bfloat16)
                        ev, od = plsc.unpack(
                            y, format=plsc.PackFormat.INTERLEAVED,
                            preferred_element_type=jnp.float32)
                        new.append(acc[2 * g] + ev)
                        new.append(acc[2 * g + 1] + od)
                    return tuple(new)

                acc = lax.fori_loop(
                    0, L, row_body,
                    tuple(jnp.zeros((16,), jnp.float32)
                          for _ in range(2 * NG)),
                    unroll=8)
                iota = jnp.arange(16, dtype=jnp.int32)
                trow = t + jnp.zeros((16,), jnp.int32)
                rlen = plsc.load_gather(rlen_v, [trow])
                for g in range(NG):
                    for r in range(2):
                        cidx = 2 * iota + (32 * g + r)
                        plsc.store_scatter(out_v, [trow, cidx],
                                           acc[2 * g + r] * rlen)

                @pl.when(t + 4 < TPB)
                def _():
                    for c in copies(t + 4, buf, sem):
                        c.start()
            return carry

        lax.fori_loop(0, TPB // 4, pair_body, 0)
        pltpu.sync_copy(out_v, out_hbm.at[pl.ds(base, TPB)])

    return sc_sum(text_flat, text_len, emb)


def _mlp_body(avg_ref, w1_ref, b1_ref, w2_ref, b2_ref, out_ref):
    h = lax.dot_general(avg_ref[...], w1_ref[...], (((1,), (1,)), ((), ())),
                        preferred_element_type=jnp.float32) + b1_ref[...]
    h = jnp.maximum(h, 0.0)
    out_ref[...] = lax.dot_general(h, w2_ref[...], (((1,), (1,)), ((), ())),
                                   preferred_element_type=jnp.float32) + b2_ref[...]


def _mlp(avg, W1, b1, W2, b2):
    B, D = avg.shape
    H = W1.shape[0]
    C = W2.shape[0]
    BT = 512
    grid = (B // BT,)
    return pl.pallas_call(
        _mlp_body,
        grid=grid,
        in_specs=[
            pl.BlockSpec((BT, D), lambda i: (i, 0)),
            pl.BlockSpec((H, D), lambda i: (0, 0)),
            pl.BlockSpec((1, H), lambda i: (0, 0)),
            pl.BlockSpec((C, H), lambda i: (0, 0)),
            pl.BlockSpec((1, C), lambda i: (0, 0)),
        ],
        out_specs=pl.BlockSpec((BT, C), lambda i: (i, 0)),
        out_shape=jax.ShapeDtypeStruct((B, C), jnp.float32),
    )(avg, W1, b1.reshape(1, H), W2, b2.reshape(1, C))


def kernel(input_text, text_len, emb, W1, b1, W2, b2):
    B, L = input_text.shape
    V, D = emb.shape
    table = _sc_transpose(emb.T).reshape(V, D // 2 + 8)
    rlen = 1.0 / text_len.astype(jnp.float32)
    avg = _sc_gather_sum(input_text.reshape(-1), rlen, table, B, L, D)
    return _mlp(avg, W1, b1, W2, b2)
